# baseline (speedup 1.0000x reference)
import jax
import jax.numpy as jnp
from jax.experimental import pallas as pl

_B, _CIN, _H, _W = 4, 3, 512, 512
_S = 16
_CO = 128
_FH, _FW = _H // _S, _W // _S
_K = _CIN * _S * _S
_HB = 32


def _patch_conv_kernel(x_ref, w_ref, b_ref, o_ref):
    w = w_ref[...]
    b = b_ref[...]
    for i in range(_HB):
        xb = x_ref[0, :, i, :, :]
        xb = xb.reshape(_CIN, _S, _FW, _S)
        xt = jnp.transpose(xb, (0, 1, 3, 2))
        xt = xt.reshape(_K, _FW)
        acc = jnp.dot(w, xt, preferred_element_type=jnp.float32)
        o_ref[0, :, i, :] = jnp.maximum(acc + b, 0.0)


def kernel(x, gts, Wc, bc):
    del gts
    xr = x.reshape(_B, _CIN, _FH, _S, _W).astype(jnp.bfloat16)
    wm = Wc.reshape(_CO, _K).astype(jnp.bfloat16)
    bm = bc.reshape(_CO, 1)
    out = pl.pallas_call(
        _patch_conv_kernel,
        grid=(_B, _FH // _HB),
        in_specs=[
            pl.BlockSpec((1, _CIN, _HB, _S, _W), lambda b, h: (b, 0, h, 0, 0)),
            pl.BlockSpec((_CO, _K), lambda b, h: (0, 0)),
            pl.BlockSpec((_CO, 1), lambda b, h: (0, 0)),
        ],
        out_specs=pl.BlockSpec((1, _CO, _HB, _FW), lambda b, h: (b, 0, h, 0)),
        out_shape=jax.ShapeDtypeStruct((_B, _CO, _FH, _FW), jnp.float32),
    )(xr, wm, bm)
    return out


# speedup vs baseline: 1.8363x; 1.8363x over previous
import jax
import jax.numpy as jnp
from jax.experimental import pallas as pl

_B, _CIN, _H, _W = 4, 3, 512, 512
_S = 16
_CO = 128
_FH, _FW = _H // _S, _W // _S
_K = _CIN * _S * _S


def _patch_conv_kernel(x_ref, e_ref, w_ref, b_ref, o_ref):
    xb = x_ref[0].reshape(_CIN * _H, _W)
    z = jnp.dot(xb, e_ref[...], preferred_element_type=jnp.float32)
    z4 = z.reshape(_CIN, _FH, _S, _W)
    v = jnp.stack([z4[:, :, :, kw * _FW:(kw + 1) * _FW] for kw in range(_S)])
    w = w_ref[...]
    b = b_ref[...]
    for h in range(_FH):
        zh = v[:, :, h].reshape(_K, _FW)
        acc = jnp.dot(w, zh, preferred_element_type=jnp.float32)
        o_ref[0, :, h, :] = jnp.maximum(acc + b, 0.0)


def kernel(x, gts, Wc, bc):
    del gts
    col = jnp.arange(_W)
    dst = (col % _S) * _FW + col // _S
    em = (dst[:, None] == jnp.arange(_W)[None, :]).astype(jnp.float32)
    wm = jnp.transpose(Wc, (3, 1, 2, 0)).reshape(_K, _CO).T
    bm = bc.reshape(_CO, 1)
    out = pl.pallas_call(
        _patch_conv_kernel,
        grid=(_B,),
        in_specs=[
            pl.BlockSpec((1, _CIN, _H, _W), lambda b: (b, 0, 0, 0)),
            pl.BlockSpec((_W, _W), lambda b: (0, 0)),
            pl.BlockSpec((_CO, _K), lambda b: (0, 0)),
            pl.BlockSpec((_CO, 1), lambda b: (0, 0)),
        ],
        out_specs=pl.BlockSpec((1, _CO, _FH, _FW), lambda b: (b, 0, 0, 0)),
        out_shape=jax.ShapeDtypeStruct((_B, _CO, _FH, _FW), jnp.float32),
    )(x, em, wm, bm)
    return out
